# baseline (speedup 1.0000x reference)
import functools

import jax
import jax.numpy as jnp
from jax import lax
from jax.experimental import pallas as pl
from jax.experimental.pallas import tpu as pltpu
from jax.experimental.pallas import tpu_sc as plsc

N = 10000
E = 320000
NPAD = 10016


def _tca_body(degout_p, degin_p, x_n, x_e, Wn1, bn1, Th1, Tb1, Ph1, Pb1,
              T1, C1, norm_s, norm_d, deg_in):
    dop = degout_p[...]
    dip = degin_p[...]
    do = (dop[0, :N] + dop[1, :N])[:, None]
    di = (dip[0, :N] + dip[1, :N])[:, None]
    deg_in[...] = di
    ns = lax.rsqrt(jnp.where(do > 0, do, 1.0))
    nd = lax.rsqrt(jnp.where(di > 0, di, 1.0))
    norm_s[...] = ns
    norm_d[...] = nd
    y1 = jnp.dot(x_n[...], Wn1[...], preferred_element_type=jnp.float32) * ns
    A1 = jnp.dot(x_e[...], Th1[...], preferred_element_type=jnp.float32)
    T1[...] = jnp.concatenate([y1, A1], axis=1)
    C1[...] = (jnp.dot(x_e[...], Ph1[...] - Th1[...],
                       preferred_element_type=jnp.float32) + Tb1[...] + Pb1[...])


def _tca(degout_p, degin_p, x_n, x_e, Wn1, bn1, Th1, Tb1, Ph1, Pb1):
    return pl.pallas_call(
        _tca_body,
        out_shape=[
            jax.ShapeDtypeStruct((N, 128), jnp.float32),
            jax.ShapeDtypeStruct((N, 64), jnp.float32),
            jax.ShapeDtypeStruct((N, 1), jnp.float32),
            jax.ShapeDtypeStruct((N, 1), jnp.float32),
            jax.ShapeDtypeStruct((N, 1), jnp.float32),
        ],
    )(degout_p, degin_p, x_n, x_e, Wn1, bn1, Th1, Tb1, Ph1, Pb1)


def _tcb_body(SM1, C1, deg_in, norm_s, norm_d, bn1, Wn2, Th2, Tb2,
              Ph2, Pb2, T2, C2):
    sm = SM1[...]
    S1 = sm[:N, :64]
    M1n = sm[:N, 64:]
    h1 = jax.nn.relu(S1 * norm_d[...] + bn1[...])
    he1 = jax.nn.relu(jnp.where(deg_in[...] > 0, M1n + C1[...], 0.0))
    y2 = jnp.dot(h1, Wn2[...], preferred_element_type=jnp.float32) * norm_s[...]
    A2 = jnp.dot(he1, Th2[...], preferred_element_type=jnp.float32)
    T2[...] = jnp.concatenate([y2, A2], axis=1)
    C2[...] = (jnp.dot(he1, Ph2[...] - Th2[...],
                       preferred_element_type=jnp.float32) + Tb2[...] + Pb2[...])


def _tcb(SM1, C1, deg_in, norm_s, norm_d, bn1, Wn2, Th2, Tb2, Ph2, Pb2):
    return pl.pallas_call(
        _tcb_body,
        out_shape=[
            jax.ShapeDtypeStruct((N, 128), jnp.float32),
            jax.ShapeDtypeStruct((N, 64), jnp.float32),
        ],
    )(SM1, C1, deg_in, norm_s, norm_d, bn1, Wn2, Th2, Tb2, Ph2, Pb2)


def _tcc_body(SM2, C2, deg_in, norm_d, bn2, h):
    sm = SM2[...]
    S2 = sm[:N, :64]
    M2n = sm[:N, 64:]
    h_n = S2 * norm_d[...] + bn2[...]
    h_e = jnp.where(deg_in[...] > 0, M2n + C2[...], 0.0)
    h[...] = jnp.concatenate([h_n, h_e], axis=1)


def _tcc(SM2, C2, deg_in, norm_d, bn2):
    return pl.pallas_call(
        _tcc_body,
        out_shape=jax.ShapeDtypeStruct((N, 128), jnp.float32),
    )(SM2, C2, deg_in, norm_d, bn2)


EC = E // 32
NB = 32
BKT = 313
RS = 10240
DEGP = 10240


def _k1_partition(src, dst):
    mesh = plsc.VectorSubcoreMesh(core_axis_name="c", subcore_axis_name="s")

    @functools.partial(
        pl.kernel,
        out_type=[
            jax.ShapeDtypeStruct((32 * RS,), jnp.int32),
            jax.ShapeDtypeStruct((32 * RS,), jnp.int32),
            jax.ShapeDtypeStruct((32 * 48,), jnp.int32),
            jax.ShapeDtypeStruct((2 * DEGP,), jnp.float32),
            jax.ShapeDtypeStruct((2 * DEGP,), jnp.float32),
        ],
        mesh=mesh,
        scratch_types=[
            pltpu.VMEM((EC,), jnp.int32),
            pltpu.VMEM((EC,), jnp.int32),
            pltpu.VMEM((512,), jnp.int32),
            pltpu.VMEM((512,), jnp.int32),
            pltpu.VMEM((NPAD,), jnp.int32),
            pltpu.VMEM((NPAD,), jnp.int32),
            pltpu.VMEM((48,), jnp.int32),
            pltpu.VMEM((EC,), jnp.float32),
            pltpu.VMEM((640,), jnp.float32),
            pltpu.VMEM_SHARED((DEGP,), jnp.float32),
            pltpu.VMEM_SHARED((DEGP,), jnp.float32),
            pltpu.SemaphoreType.DMA,
            pltpu.SemaphoreType.DMA,
        ],
        compiler_params=pltpu.CompilerParams(needs_layout_passes=False),
    )
    def k(src_hbm, dst_hbm, psrc_hbm, pdst_hbm, offs_hbm, degin_hbm, degout_hbm,
          src_v, dst_v, cnt, cur, ssrc, sdst, offs_v, ones, zbuf,
          degin_sh, degout_sh, sem1, sem2):
        cid = lax.axis_index("c")
        sid = lax.axis_index("s")
        wid = sid * 2 + cid
        lanes = lax.iota(jnp.int32, 16)

        pltpu.sync_copy(src_hbm.at[pl.ds(wid * EC, EC)], src_v)
        pltpu.sync_copy(dst_hbm.at[pl.ds(wid * EC, EC)], dst_v)

        def fz(i, _):
            zbuf[pl.ds(i * 16, 16)] = jnp.zeros((16,), jnp.float32)
            return 0
        lax.fori_loop(0, 40, fz, 0)

        def fo(i, _):
            ones[pl.ds(i * 16, 16)] = jnp.ones((16,), jnp.float32)
            return 0
        lax.fori_loop(0, EC // 16, fo, 0)

        def fc(i, _):
            cnt[pl.ds(i * 16, 16)] = jnp.zeros((16,), jnp.int32)
            return 0
        lax.fori_loop(0, 32, fc, 0)

        pltpu.sync_copy(zbuf, degin_sh.at[pl.ds(sid * 640, 640)])
        pltpu.sync_copy(zbuf, degout_sh.at[pl.ds(sid * 640, 640)])
        plsc.subcore_barrier()

        cp_in = pltpu.async_copy(ones, degin_sh.at[dst_v], sem1, add=True)
        cp_out = pltpu.async_copy(ones, degout_sh.at[src_v], sem2, add=True)

        def hist(gi, _):
            dvec = dst_v[pl.ds(gi * 16, 16)]
            idx = (dvec // BKT) * 16 + lanes
            c = plsc.load_gather(cnt, [idx])
            plsc.store_scatter(cnt, [idx], c + 1)
            return 0
        lax.fori_loop(0, EC // 16, hist, 0)

        def scan(b2, total):
            row = cnt[pl.ds(b2 * 16, 16)]
            exc = plsc.cumsum(row) - row
            cur[pl.ds(b2 * 16, 16)] = exc + total
            return total + jnp.sum(row)
        lax.fori_loop(0, 32, scan, jnp.int32(0))

        offs_v[pl.ds(0, 16)] = plsc.load_gather(cur, [lanes * 16])
        offs_v[pl.ds(16, 16)] = plsc.load_gather(cur, [(lanes + 16) * 16])
        offs_v[pl.ds(32, 16)] = jnp.where(lanes == 0, EC, 0).astype(jnp.int32)

        def scat(gi, _):
            dvec = dst_v[pl.ds(gi * 16, 16)]
            svec = src_v[pl.ds(gi * 16, 16)]
            idx = (dvec // BKT) * 16 + lanes
            pos = plsc.load_gather(cur, [idx])
            plsc.store_scatter(cur, [idx], pos + 1)
            plsc.store_scatter(ssrc, [pos], svec)
            plsc.store_scatter(sdst, [pos], dvec)
            return 0
        lax.fori_loop(0, EC // 16, scat, 0)

        pltpu.sync_copy(ssrc, psrc_hbm.at[pl.ds(wid * RS, NPAD)])
        pltpu.sync_copy(sdst, pdst_hbm.at[pl.ds(wid * RS, NPAD)])
        pltpu.sync_copy(offs_v, offs_hbm.at[pl.ds(wid * 48, 48)])

        cp_in.wait()
        cp_out.wait()
        plsc.subcore_barrier()

        @pl.when(sid == 0)
        def _():
            pltpu.sync_copy(degin_sh, degin_hbm.at[pl.ds(cid * DEGP, DEGP)])
            pltpu.sync_copy(degout_sh, degout_hbm.at[pl.ds(cid * DEGP, DEGP)])

    return k(src, dst)


def _k2_segpass(psrc, pdst, offs, T):
    CH = 512
    mesh = plsc.VectorSubcoreMesh(core_axis_name="c", subcore_axis_name="s")

    @functools.partial(
        pl.kernel,
        out_type=jax.ShapeDtypeStruct((NPAD * 128,), jnp.float32),
        mesh=mesh,
        scratch_types=[
            pltpu.VMEM((32 * 48,), jnp.int32),
            pltpu.VMEM((314 * 128,), jnp.float32),
            pltpu.VMEM((CH,), jnp.int32),
            pltpu.VMEM((CH,), jnp.int32),
            pltpu.VMEM((CH + 16,), jnp.int32),
            pltpu.VMEM((CH, 128), jnp.float32),
            pltpu.SemaphoreType.DMA,
        ],
        compiler_params=pltpu.CompilerParams(needs_layout_passes=False),
    )
    def k(psrc_hbm, pdst_hbm, offs_hbm, T_hbm, SM_hbm,
          offs_v, acc, csrc, cdst, crow, rows, sem):
        cid = lax.axis_index("c")
        sid = lax.axis_index("s")
        wid = sid * 2 + cid
        b = wid
        lanes = lax.iota(jnp.int32, 16)
        neg = jnp.full((16,), -jnp.inf, jnp.float32)
        zero = jnp.zeros((16,), jnp.float32)

        pltpu.sync_copy(offs_hbm, offs_v)

        def fa(i, _):
            v = jnp.where((i % 8) < 4, zero, neg)
            acc[pl.ds(i * 16, 16)] = v
            return 0
        lax.fori_loop(0, 314 * 8, fa, 0)

        def seg(t2, _):
            v = offs_v[pl.ds(t2 * 48 + b, 16)]
            start = v[0]
            end = v[1]
            astart = (start // 8) * 8
            s0 = start - astart
            total = s0 + (end - start)
            nch = (total + CH - 1) // CH

            def chunk(j, _):
                off = t2 * RS + astart + j * CH
                pltpu.sync_copy(psrc_hbm.at[pl.ds(off, CH)], csrc)
                pltpu.sync_copy(pdst_hbm.at[pl.ds(off, CH)], cdst)

                def san(gi, _):
                    p = j * CH + gi * 16 + lanes
                    valid = (p >= s0) & (p < total)
                    sv = csrc[pl.ds(gi * 16, 16)]
                    dv = cdst[pl.ds(gi * 16, 16)]
                    csrc[pl.ds(gi * 16, 16)] = jnp.where(valid, sv, 0)
                    crow[pl.ds(gi * 16, 16)] = jnp.where(
                        valid, dv - b * BKT, 313)
                    return 0
                lax.fori_loop(0, CH // 16, san, 0)

                pltpu.async_copy(T_hbm.at[csrc], rows, sem).wait()

                hi = jnp.minimum(total - j * CH, CH)

                def ap(e, _):
                    rv = crow[pl.ds(e, 16)]
                    ra = rv[0] * 128
                    s0_ = acc[pl.ds(ra, 16)]
                    s1_ = acc[pl.ds(ra + 16, 16)]
                    s2_ = acc[pl.ds(ra + 32, 16)]
                    s3_ = acc[pl.ds(ra + 48, 16)]
                    acc[pl.ds(ra, 16)] = s0_ + rows[e, pl.ds(0, 16)]
                    acc[pl.ds(ra + 16, 16)] = s1_ + rows[e, pl.ds(16, 16)]
                    acc[pl.ds(ra + 32, 16)] = s2_ + rows[e, pl.ds(32, 16)]
                    acc[pl.ds(ra + 48, 16)] = s3_ + rows[e, pl.ds(48, 16)]
                    m0 = acc[pl.ds(ra + 64, 16)]
                    m1 = acc[pl.ds(ra + 80, 16)]
                    m2 = acc[pl.ds(ra + 96, 16)]
                    m3 = acc[pl.ds(ra + 112, 16)]
                    acc[pl.ds(ra + 64, 16)] = jnp.maximum(m0, rows[e, pl.ds(64, 16)])
                    acc[pl.ds(ra + 80, 16)] = jnp.maximum(m1, rows[e, pl.ds(80, 16)])
                    acc[pl.ds(ra + 96, 16)] = jnp.maximum(m2, rows[e, pl.ds(96, 16)])
                    acc[pl.ds(ra + 112, 16)] = jnp.maximum(m3, rows[e, pl.ds(112, 16)])
                    return 0
                lax.fori_loop(0, hi, ap, 0)
                return 0

            lax.fori_loop(0, nch, chunk, 0)
            return 0

        lax.fori_loop(0, 32, seg, 0)

        pltpu.sync_copy(acc.at[pl.ds(0, BKT * 128)],
                        SM_hbm.at[pl.ds(b * BKT * 128, BKT * 128)])

    return k(psrc, pdst, offs, T)


def _k3_dot(src, dst, h):
    CH = 80
    NCH = EC // CH
    mesh = plsc.VectorSubcoreMesh(core_axis_name="c", subcore_axis_name="s")

    @functools.partial(
        pl.kernel,
        out_type=jax.ShapeDtypeStruct((E,), jnp.float32),
        mesh=mesh,
        scratch_types=[
            pltpu.VMEM((CH,), jnp.int32),
            pltpu.VMEM((CH,), jnp.int32),
            pltpu.VMEM((CH,), jnp.int32),
            pltpu.VMEM((CH,), jnp.int32),
            pltpu.VMEM((CH, 128), jnp.float32),
            pltpu.VMEM((CH, 128), jnp.float32),
            pltpu.VMEM((CH, 128), jnp.float32),
            pltpu.VMEM((CH, 128), jnp.float32),
            pltpu.VMEM((CH,), jnp.float32),
            pltpu.SemaphoreType.DMA,
            pltpu.SemaphoreType.DMA,
            pltpu.SemaphoreType.DMA,
            pltpu.SemaphoreType.DMA,
        ],
        compiler_params=pltpu.CompilerParams(needs_layout_passes=False),
    )
    def k(src_hbm, dst_hbm, h_hbm, out_hbm, csrc0, csrc1, cdst0, cdst1,
          U0, U1, V0, V1, outv, sU0, sU1, sV0, sV1):
        cid = lax.axis_index("c")
        sid = lax.axis_index("s")
        wid = sid * 2 + cid
        base = wid * EC
        lanes = lax.iota(jnp.int32, 16)
        CS = (csrc0, csrc1)
        CD = (cdst0, cdst1)
        UU = (U0, U1)
        VV = (V0, V1)
        semU = (sU0, sU1)
        semV = (sV0, sV1)

        def issue(j, slot):
            pltpu.sync_copy(src_hbm.at[pl.ds(base + j * CH, CH)], CS[slot])
            pltpu.sync_copy(dst_hbm.at[pl.ds(base + j * CH, CH)], CD[slot])
            pltpu.async_copy(h_hbm.at[CS[slot]], UU[slot], semU[slot])
            pltpu.async_copy(h_hbm.at[CD[slot]], VV[slot], semV[slot])

        def wait(slot):
            pltpu.make_async_copy(h_hbm.at[CS[slot]], UU[slot], semU[slot]).wait()
            pltpu.make_async_copy(h_hbm.at[CD[slot]], VV[slot], semV[slot]).wait()

        def compute(j, slot):
            Ur, Vr = UU[slot], VV[slot]

            def group(gi, _):
                res = jnp.zeros((16,), jnp.float32)
                for i in range(16):
                    e = gi * 16 + i
                    d = Ur[e, pl.ds(0, 16)] * Vr[e, pl.ds(0, 16)]
                    for c in range(1, 8):
                        d = d + Ur[e, pl.ds(c * 16, 16)] * Vr[e, pl.ds(c * 16, 16)]
                    res = jnp.where(lanes == i, jnp.sum(d), res)
                outv[pl.ds(gi * 16, 16)] = res
                return 0

            lax.fori_loop(0, CH // 16, group, 0)
            pltpu.sync_copy(outv, out_hbm.at[pl.ds(base + j * CH, CH)])

        issue(0, 0)

        def pair(i, _):
            j0 = 2 * i
            wait(0)
            issue(j0 + 1, 1)
            compute(j0, 0)
            wait(1)

            @pl.when(j0 + 2 < NCH)
            def _():
                issue(j0 + 2, 0)

            compute(j0 + 1, 1)
            return 0

        lax.fori_loop(0, (NCH - 1) // 2, pair, 0)
        wait(0)
        compute(NCH - 1, 0)

    return k(src, dst, h)


def kernel(g, x_n, x_e, Wn1, bn1, Wn2, bn2, Th1, Tb1, Ph1, Pb1, Th2, Tb2, Ph2, Pb2):
    src = g[0].reshape(E)
    dst = g[1].reshape(E)

    psrc, pdst, offs, degin_p, degout_p = _k1_partition(src, dst)

    T1, C1, norm_s, norm_d, deg_in = _tca(
        degout_p.reshape(2, DEGP), degin_p.reshape(2, DEGP), x_n, x_e, Wn1,
        bn1.reshape(1, 64), Th1, Tb1.reshape(1, 64), Ph1, Pb1.reshape(1, 64))

    SM1 = _k2_segpass(psrc, pdst, offs, T1).reshape(NPAD, 128)

    T2, C2 = _tcb(SM1, C1, deg_in, norm_s, norm_d, bn1.reshape(1, 64), Wn2,
                  Th2, Tb2.reshape(1, 64), Ph2, Pb2.reshape(1, 64))

    SM2 = _k2_segpass(psrc, pdst, offs, T2).reshape(NPAD, 128)

    h = _tcc(SM2, C2, deg_in, norm_d, bn2.reshape(1, 64))

    score = _k3_dot(src, dst, h)
    return score.reshape(E, 1)


# speedup vs baseline: 1.0013x; 1.0013x over previous
import functools

import jax
import jax.numpy as jnp
from jax import lax
from jax.experimental import pallas as pl
from jax.experimental.pallas import tpu as pltpu
from jax.experimental.pallas import tpu_sc as plsc

N = 10000
E = 320000
NPAD = 10016


def _tca_body(degout_p, degin_p, x_n, x_e, Wn1, bn1, Th1, Tb1, Ph1, Pb1,
              T1, C1, norm_s, norm_d, deg_in):
    dop = degout_p[...]
    dip = degin_p[...]
    do = (dop[0, :N] + dop[1, :N])[:, None]
    di = (dip[0, :N] + dip[1, :N])[:, None]
    deg_in[...] = di
    ns = lax.rsqrt(jnp.where(do > 0, do, 1.0))
    nd = lax.rsqrt(jnp.where(di > 0, di, 1.0))
    norm_s[...] = ns
    norm_d[...] = nd
    y1 = jnp.dot(x_n[...], Wn1[...], preferred_element_type=jnp.float32) * ns
    A1 = jnp.dot(x_e[...], Th1[...], preferred_element_type=jnp.float32)
    T1[...] = jnp.concatenate([y1, A1], axis=1)
    C1[...] = (jnp.dot(x_e[...], Ph1[...] - Th1[...],
                       preferred_element_type=jnp.float32) + Tb1[...] + Pb1[...])


def _tca(degout_p, degin_p, x_n, x_e, Wn1, bn1, Th1, Tb1, Ph1, Pb1):
    return pl.pallas_call(
        _tca_body,
        out_shape=[
            jax.ShapeDtypeStruct((N, 128), jnp.float32),
            jax.ShapeDtypeStruct((N, 64), jnp.float32),
            jax.ShapeDtypeStruct((N, 1), jnp.float32),
            jax.ShapeDtypeStruct((N, 1), jnp.float32),
            jax.ShapeDtypeStruct((N, 1), jnp.float32),
        ],
    )(degout_p, degin_p, x_n, x_e, Wn1, bn1, Th1, Tb1, Ph1, Pb1)


def _tcb_body(SM1, C1, deg_in, norm_s, norm_d, bn1, Wn2, Th2, Tb2,
              Ph2, Pb2, T2, C2):
    sm = SM1[...]
    S1 = sm[:N, :64]
    M1n = sm[:N, 64:]
    h1 = jax.nn.relu(S1 * norm_d[...] + bn1[...])
    he1 = jax.nn.relu(jnp.where(deg_in[...] > 0, M1n + C1[...], 0.0))
    y2 = jnp.dot(h1, Wn2[...], preferred_element_type=jnp.float32) * norm_s[...]
    A2 = jnp.dot(he1, Th2[...], preferred_element_type=jnp.float32)
    T2[...] = jnp.concatenate([y2, A2], axis=1)
    C2[...] = (jnp.dot(he1, Ph2[...] - Th2[...],
                       preferred_element_type=jnp.float32) + Tb2[...] + Pb2[...])


def _tcb(SM1, C1, deg_in, norm_s, norm_d, bn1, Wn2, Th2, Tb2, Ph2, Pb2):
    return pl.pallas_call(
        _tcb_body,
        out_shape=[
            jax.ShapeDtypeStruct((N, 128), jnp.float32),
            jax.ShapeDtypeStruct((N, 64), jnp.float32),
        ],
    )(SM1, C1, deg_in, norm_s, norm_d, bn1, Wn2, Th2, Tb2, Ph2, Pb2)


def _tcc_body(SM2, C2, deg_in, norm_d, bn2, h):
    sm = SM2[...]
    S2 = sm[:N, :64]
    M2n = sm[:N, 64:]
    h_n = S2 * norm_d[...] + bn2[...]
    h_e = jnp.where(deg_in[...] > 0, M2n + C2[...], 0.0)
    h[...] = jnp.concatenate([h_n, h_e], axis=1)


def _tcc(SM2, C2, deg_in, norm_d, bn2):
    return pl.pallas_call(
        _tcc_body,
        out_shape=jax.ShapeDtypeStruct((N, 128), jnp.float32),
    )(SM2, C2, deg_in, norm_d, bn2)


EC = E // 32
NB = 32
BKT = 313
RS = 10240
DEGP = 10240


def _k1_partition(src, dst):
    mesh = plsc.VectorSubcoreMesh(core_axis_name="c", subcore_axis_name="s")

    @functools.partial(
        pl.kernel,
        out_type=[
            jax.ShapeDtypeStruct((32 * RS,), jnp.int32),
            jax.ShapeDtypeStruct((32 * RS,), jnp.int32),
            jax.ShapeDtypeStruct((32 * 48,), jnp.int32),
            jax.ShapeDtypeStruct((2 * DEGP,), jnp.float32),
            jax.ShapeDtypeStruct((2 * DEGP,), jnp.float32),
        ],
        mesh=mesh,
        scratch_types=[
            pltpu.VMEM((EC,), jnp.int32),
            pltpu.VMEM((EC,), jnp.int32),
            pltpu.VMEM((512,), jnp.int32),
            pltpu.VMEM((512,), jnp.int32),
            pltpu.VMEM((NPAD,), jnp.int32),
            pltpu.VMEM((NPAD,), jnp.int32),
            pltpu.VMEM((48,), jnp.int32),
            pltpu.VMEM((EC,), jnp.float32),
            pltpu.VMEM((640,), jnp.float32),
            pltpu.VMEM_SHARED((DEGP,), jnp.float32),
            pltpu.VMEM_SHARED((DEGP,), jnp.float32),
            pltpu.SemaphoreType.DMA,
            pltpu.SemaphoreType.DMA,
        ],
        compiler_params=pltpu.CompilerParams(needs_layout_passes=False),
    )
    def k(src_hbm, dst_hbm, psrc_hbm, pdst_hbm, offs_hbm, degin_hbm, degout_hbm,
          src_v, dst_v, cnt, cur, ssrc, sdst, offs_v, ones, zbuf,
          degin_sh, degout_sh, sem1, sem2):
        cid = lax.axis_index("c")
        sid = lax.axis_index("s")
        wid = sid * 2 + cid
        lanes = lax.iota(jnp.int32, 16)

        pltpu.sync_copy(src_hbm.at[pl.ds(wid * EC, EC)], src_v)
        pltpu.sync_copy(dst_hbm.at[pl.ds(wid * EC, EC)], dst_v)

        def fz(i, _):
            zbuf[pl.ds(i * 16, 16)] = jnp.zeros((16,), jnp.float32)
            return 0
        lax.fori_loop(0, 40, fz, 0)

        def fo(i, _):
            ones[pl.ds(i * 16, 16)] = jnp.ones((16,), jnp.float32)
            return 0
        lax.fori_loop(0, EC // 16, fo, 0)

        def fc(i, _):
            cnt[pl.ds(i * 16, 16)] = jnp.zeros((16,), jnp.int32)
            return 0
        lax.fori_loop(0, 32, fc, 0)

        pltpu.sync_copy(zbuf, degin_sh.at[pl.ds(sid * 640, 640)])
        pltpu.sync_copy(zbuf, degout_sh.at[pl.ds(sid * 640, 640)])
        plsc.subcore_barrier()

        cp_in = pltpu.async_copy(ones, degin_sh.at[dst_v], sem1, add=True)
        cp_out = pltpu.async_copy(ones, degout_sh.at[src_v], sem2, add=True)

        def hist(gi, _):
            dvec = dst_v[pl.ds(gi * 16, 16)]
            idx = (dvec // BKT) * 16 + lanes
            c = plsc.load_gather(cnt, [idx])
            plsc.store_scatter(cnt, [idx], c + 1)
            return 0
        lax.fori_loop(0, EC // 16, hist, 0)

        def scan(b2, total):
            row = cnt[pl.ds(b2 * 16, 16)]
            exc = plsc.cumsum(row) - row
            cur[pl.ds(b2 * 16, 16)] = exc + total
            return total + jnp.sum(row)
        lax.fori_loop(0, 32, scan, jnp.int32(0))

        offs_v[pl.ds(0, 16)] = plsc.load_gather(cur, [lanes * 16])
        offs_v[pl.ds(16, 16)] = plsc.load_gather(cur, [(lanes + 16) * 16])
        offs_v[pl.ds(32, 16)] = jnp.where(lanes == 0, EC, 0).astype(jnp.int32)

        def scat(gi, _):
            dvec = dst_v[pl.ds(gi * 16, 16)]
            svec = src_v[pl.ds(gi * 16, 16)]
            idx = (dvec // BKT) * 16 + lanes
            pos = plsc.load_gather(cur, [idx])
            plsc.store_scatter(cur, [idx], pos + 1)
            plsc.store_scatter(ssrc, [pos], svec)
            plsc.store_scatter(sdst, [pos], dvec)
            return 0
        lax.fori_loop(0, EC // 16, scat, 0)

        pltpu.sync_copy(ssrc, psrc_hbm.at[pl.ds(wid * RS, NPAD)])
        pltpu.sync_copy(sdst, pdst_hbm.at[pl.ds(wid * RS, NPAD)])
        pltpu.sync_copy(offs_v, offs_hbm.at[pl.ds(wid * 48, 48)])

        cp_in.wait()
        cp_out.wait()
        plsc.subcore_barrier()

        @pl.when(sid == 0)
        def _():
            pltpu.sync_copy(degin_sh, degin_hbm.at[pl.ds(cid * DEGP, DEGP)])
            pltpu.sync_copy(degout_sh, degout_hbm.at[pl.ds(cid * DEGP, DEGP)])

    return k(src, dst)


def _k2_segpass(psrc, pdst, offs, T):
    CH = 512
    mesh = plsc.VectorSubcoreMesh(core_axis_name="c", subcore_axis_name="s")

    @functools.partial(
        pl.kernel,
        out_type=jax.ShapeDtypeStruct((NPAD * 128,), jnp.float32),
        mesh=mesh,
        scratch_types=[
            pltpu.VMEM((32 * 48,), jnp.int32),
            pltpu.VMEM((330 * 128,), jnp.float32),
            pltpu.VMEM((CH,), jnp.int32),
            pltpu.VMEM((CH,), jnp.int32),
            pltpu.VMEM((CH + 16,), jnp.int32),
            pltpu.VMEM((CH, 128), jnp.float32),
            pltpu.VMEM((336,), jnp.int32),
            pltpu.SemaphoreType.DMA,
        ],
        compiler_params=pltpu.CompilerParams(needs_layout_passes=False),
    )
    def k(psrc_hbm, pdst_hbm, offs_hbm, T_hbm, SM_hbm,
          offs_v, acc, csrc, cdst, crow, rows, tag, sem):
        cid = lax.axis_index("c")
        sid = lax.axis_index("s")
        wid = sid * 2 + cid
        b = wid
        lanes = lax.iota(jnp.int32, 16)
        neg = jnp.full((16,), -jnp.inf, jnp.float32)
        zero = jnp.zeros((16,), jnp.float32)
        cvecs = [c * 16 + lanes for c in range(8)]

        pltpu.sync_copy(offs_hbm, offs_v)

        def fa(i, _):
            v = jnp.where((i % 8) < 4, zero, neg)
            acc[pl.ds(i * 16, 16)] = v
            return 0
        lax.fori_loop(0, 330 * 8, fa, 0)

        def seg(t2, _):
            v = offs_v[pl.ds(t2 * 48 + b, 16)]
            start = v[0]
            end = v[1]
            astart = (start // 8) * 8
            s0 = start - astart
            total = s0 + (end - start)
            nch = (total + CH - 1) // CH

            def chunk(j, _):
                off = t2 * RS + astart + j * CH
                pltpu.sync_copy(psrc_hbm.at[pl.ds(off, CH)], csrc)
                pltpu.sync_copy(pdst_hbm.at[pl.ds(off, CH)], cdst)

                def san(gi, _):
                    p = j * CH + gi * 16 + lanes
                    valid = (p >= s0) & (p < total)
                    sv = csrc[pl.ds(gi * 16, 16)]
                    dv = cdst[pl.ds(gi * 16, 16)]
                    csrc[pl.ds(gi * 16, 16)] = jnp.where(valid, sv, 0)
                    crow[pl.ds(gi * 16, 16)] = jnp.where(
                        valid, dv - b * BKT, 313 + (p & 15))
                    return 0
                lax.fori_loop(0, CH // 16, san, 0)

                pltpu.async_copy(T_hbm.at[csrc], rows, sem).wait()

                def ap(gi, _):
                    gvec = crow[pl.ds(gi * 16, 16)]
                    evec = gi * 16 + lanes
                    ra = gvec * 128
                    vals = [plsc.load_gather(rows, [evec, cvecs[c]])
                            for c in range(8)]

                    def cond(committed):
                        return jnp.sum(jnp.where(committed, 0, 1)) > 0

                    def body(committed):
                        todo = jnp.logical_not(committed)
                        plsc.store_scatter(tag, [gvec], lanes, mask=todo)
                        w = plsc.load_gather(tag, [gvec])
                        winner = (w == lanes) & todo
                        for c in range(8):
                            idx = ra + cvecs[c]
                            cur = plsc.load_gather(acc, [idx])
                            new = (cur + vals[c]) if c < 4 \
                                else jnp.maximum(cur, vals[c])
                            plsc.store_scatter(acc, [idx], new, mask=winner)
                        return committed | winner

                    lax.while_loop(cond, body, jnp.zeros((16,), jnp.bool_))
                    return 0
                lax.fori_loop(0, CH // 16, ap, 0)
                return 0

            lax.fori_loop(0, nch, chunk, 0)
            return 0

        lax.fori_loop(0, 32, seg, 0)

        pltpu.sync_copy(acc.at[pl.ds(0, BKT * 128)],
                        SM_hbm.at[pl.ds(b * BKT * 128, BKT * 128)])

    return k(psrc, pdst, offs, T)


def _k3_dot(src, dst, h):
    CH = 80
    NCH = EC // CH
    mesh = plsc.VectorSubcoreMesh(core_axis_name="c", subcore_axis_name="s")

    @functools.partial(
        pl.kernel,
        out_type=jax.ShapeDtypeStruct((E,), jnp.float32),
        mesh=mesh,
        scratch_types=[
            pltpu.VMEM((CH,), jnp.int32),
            pltpu.VMEM((CH,), jnp.int32),
            pltpu.VMEM((CH,), jnp.int32),
            pltpu.VMEM((CH,), jnp.int32),
            pltpu.VMEM((CH, 128), jnp.float32),
            pltpu.VMEM((CH, 128), jnp.float32),
            pltpu.VMEM((CH, 128), jnp.float32),
            pltpu.VMEM((CH, 128), jnp.float32),
            pltpu.VMEM((CH,), jnp.float32),
            pltpu.SemaphoreType.DMA,
            pltpu.SemaphoreType.DMA,
            pltpu.SemaphoreType.DMA,
            pltpu.SemaphoreType.DMA,
        ],
        compiler_params=pltpu.CompilerParams(needs_layout_passes=False),
    )
    def k(src_hbm, dst_hbm, h_hbm, out_hbm, csrc0, csrc1, cdst0, cdst1,
          U0, U1, V0, V1, outv, sU0, sU1, sV0, sV1):
        cid = lax.axis_index("c")
        sid = lax.axis_index("s")
        wid = sid * 2 + cid
        base = wid * EC
        lanes = lax.iota(jnp.int32, 16)
        CS = (csrc0, csrc1)
        CD = (cdst0, cdst1)
        UU = (U0, U1)
        VV = (V0, V1)
        semU = (sU0, sU1)
        semV = (sV0, sV1)

        def issue(j, slot):
            pltpu.sync_copy(src_hbm.at[pl.ds(base + j * CH, CH)], CS[slot])
            pltpu.sync_copy(dst_hbm.at[pl.ds(base + j * CH, CH)], CD[slot])
            pltpu.async_copy(h_hbm.at[CS[slot]], UU[slot], semU[slot])
            pltpu.async_copy(h_hbm.at[CD[slot]], VV[slot], semV[slot])

        def wait(slot):
            pltpu.make_async_copy(h_hbm.at[CS[slot]], UU[slot], semU[slot]).wait()
            pltpu.make_async_copy(h_hbm.at[CD[slot]], VV[slot], semV[slot]).wait()

        def compute(j, slot):
            Ur, Vr = UU[slot], VV[slot]

            def group(gi, _):
                res = jnp.zeros((16,), jnp.float32)
                for i in range(16):
                    e = gi * 16 + i
                    d = Ur[e, pl.ds(0, 16)] * Vr[e, pl.ds(0, 16)]
                    for c in range(1, 8):
                        d = d + Ur[e, pl.ds(c * 16, 16)] * Vr[e, pl.ds(c * 16, 16)]
                    res = jnp.where(lanes == i, jnp.sum(d), res)
                outv[pl.ds(gi * 16, 16)] = res
                return 0

            lax.fori_loop(0, CH // 16, group, 0)
            pltpu.sync_copy(outv, out_hbm.at[pl.ds(base + j * CH, CH)])

        issue(0, 0)

        def pair(i, _):
            j0 = 2 * i
            wait(0)
            issue(j0 + 1, 1)
            compute(j0, 0)
            wait(1)

            @pl.when(j0 + 2 < NCH)
            def _():
                issue(j0 + 2, 0)

            compute(j0 + 1, 1)
            return 0

        lax.fori_loop(0, (NCH - 1) // 2, pair, 0)
        wait(0)
        compute(NCH - 1, 0)

    return k(src, dst, h)


def kernel(g, x_n, x_e, Wn1, bn1, Wn2, bn2, Th1, Tb1, Ph1, Pb1, Th2, Tb2, Ph2, Pb2):
    src = g[0].reshape(E)
    dst = g[1].reshape(E)

    psrc, pdst, offs, degin_p, degout_p = _k1_partition(src, dst)

    T1, C1, norm_s, norm_d, deg_in = _tca(
        degout_p.reshape(2, DEGP), degin_p.reshape(2, DEGP), x_n, x_e, Wn1,
        bn1.reshape(1, 64), Th1, Tb1.reshape(1, 64), Ph1, Pb1.reshape(1, 64))

    SM1 = _k2_segpass(psrc, pdst, offs, T1).reshape(NPAD, 128)

    T2, C2 = _tcb(SM1, C1, deg_in, norm_s, norm_d, bn1.reshape(1, 64), Wn2,
                  Th2, Tb2.reshape(1, 64), Ph2, Pb2.reshape(1, 64))

    SM2 = _k2_segpass(psrc, pdst, offs, T2).reshape(NPAD, 128)

    h = _tcc(SM2, C2, deg_in, norm_d, bn2.reshape(1, 64))

    score = _k3_dot(src, dst, h)
    return score.reshape(E, 1)


# speedup vs baseline: 14.6856x; 14.6664x over previous
import functools

import jax
import jax.numpy as jnp
from jax import lax
from jax.experimental import pallas as pl
from jax.experimental.pallas import tpu as pltpu
from jax.experimental.pallas import tpu_sc as plsc

N = 10000
E = 320000
NPAD = 10016


def _tca_body(degout_p, degin_p, x_n, x_e, Wn1, bn1, Th1, Tb1, Ph1, Pb1,
              T1, C1, norm_s, norm_d, deg_in):
    dop = degout_p[...]
    dip = degin_p[...]
    do = (dop[0, :N] + dop[1, :N])[:, None]
    di = (dip[0, :N] + dip[1, :N])[:, None]
    deg_in[...] = di
    ns = lax.rsqrt(jnp.where(do > 0, do, 1.0))
    nd = lax.rsqrt(jnp.where(di > 0, di, 1.0))
    norm_s[...] = ns
    norm_d[...] = nd
    y1 = jnp.dot(x_n[...], Wn1[...], preferred_element_type=jnp.float32) * ns
    A1 = jnp.dot(x_e[...], Th1[...], preferred_element_type=jnp.float32)
    T1[...] = jnp.concatenate([y1, A1], axis=1)
    C1[...] = (jnp.dot(x_e[...], Ph1[...] - Th1[...],
                       preferred_element_type=jnp.float32) + Tb1[...] + Pb1[...])


def _tca(degout_p, degin_p, x_n, x_e, Wn1, bn1, Th1, Tb1, Ph1, Pb1):
    return pl.pallas_call(
        _tca_body,
        out_shape=[
            jax.ShapeDtypeStruct((N, 128), jnp.float32),
            jax.ShapeDtypeStruct((N, 64), jnp.float32),
            jax.ShapeDtypeStruct((N, 1), jnp.float32),
            jax.ShapeDtypeStruct((N, 1), jnp.float32),
            jax.ShapeDtypeStruct((N, 1), jnp.float32),
        ],
    )(degout_p, degin_p, x_n, x_e, Wn1, bn1, Th1, Tb1, Ph1, Pb1)


def _tcb_body(SM1, C1, deg_in, norm_s, norm_d, bn1, Wn2, Th2, Tb2,
              Ph2, Pb2, T2, C2):
    sm = SM1[...]
    S1 = sm[:N, :64]
    M1n = sm[:N, 64:]
    h1 = jax.nn.relu(S1 * norm_d[...] + bn1[...])
    he1 = jax.nn.relu(jnp.where(deg_in[...] > 0, M1n + C1[...], 0.0))
    y2 = jnp.dot(h1, Wn2[...], preferred_element_type=jnp.float32) * norm_s[...]
    A2 = jnp.dot(he1, Th2[...], preferred_element_type=jnp.float32)
    T2[...] = jnp.concatenate([y2, A2], axis=1)
    C2[...] = (jnp.dot(he1, Ph2[...] - Th2[...],
                       preferred_element_type=jnp.float32) + Tb2[...] + Pb2[...])


def _tcb(SM1, C1, deg_in, norm_s, norm_d, bn1, Wn2, Th2, Tb2, Ph2, Pb2):
    return pl.pallas_call(
        _tcb_body,
        out_shape=[
            jax.ShapeDtypeStruct((N, 128), jnp.float32),
            jax.ShapeDtypeStruct((N, 64), jnp.float32),
        ],
    )(SM1, C1, deg_in, norm_s, norm_d, bn1, Wn2, Th2, Tb2, Ph2, Pb2)


def _tcc_body(SM2, C2, deg_in, norm_d, bn2, h):
    sm = SM2[...]
    S2 = sm[:N, :64]
    M2n = sm[:N, 64:]
    h_n = S2 * norm_d[...] + bn2[...]
    h_e = jnp.where(deg_in[...] > 0, M2n + C2[...], 0.0)
    h[...] = jnp.concatenate([h_n, h_e], axis=1)


def _tcc(SM2, C2, deg_in, norm_d, bn2):
    return pl.pallas_call(
        _tcc_body,
        out_shape=jax.ShapeDtypeStruct((N, 128), jnp.float32),
    )(SM2, C2, deg_in, norm_d, bn2)


EC = E // 32
NB = 32
BKT = 313
RS = 10240
DEGP = 10240


def _k1_partition(src, dst):
    mesh = plsc.VectorSubcoreMesh(core_axis_name="c", subcore_axis_name="s")

    @functools.partial(
        pl.kernel,
        out_type=[
            jax.ShapeDtypeStruct((32 * RS,), jnp.int32),
            jax.ShapeDtypeStruct((32 * RS,), jnp.int32),
            jax.ShapeDtypeStruct((32 * 48,), jnp.int32),
            jax.ShapeDtypeStruct((2 * DEGP,), jnp.float32),
            jax.ShapeDtypeStruct((2 * DEGP,), jnp.float32),
        ],
        mesh=mesh,
        scratch_types=[
            pltpu.VMEM((EC,), jnp.int32),
            pltpu.VMEM((EC,), jnp.int32),
            pltpu.VMEM((512,), jnp.int32),
            pltpu.VMEM((512,), jnp.int32),
            pltpu.VMEM((NPAD,), jnp.int32),
            pltpu.VMEM((NPAD,), jnp.int32),
            pltpu.VMEM((48,), jnp.int32),
            pltpu.VMEM((EC,), jnp.float32),
            pltpu.VMEM((640,), jnp.float32),
            pltpu.VMEM_SHARED((DEGP,), jnp.float32),
            pltpu.VMEM_SHARED((DEGP,), jnp.float32),
            pltpu.SemaphoreType.DMA,
            pltpu.SemaphoreType.DMA,
        ],
        compiler_params=pltpu.CompilerParams(needs_layout_passes=False),
    )
    def k(src_hbm, dst_hbm, psrc_hbm, pdst_hbm, offs_hbm, degin_hbm, degout_hbm,
          src_v, dst_v, cnt, cur, ssrc, sdst, offs_v, ones, zbuf,
          degin_sh, degout_sh, sem1, sem2):
        cid = lax.axis_index("c")
        sid = lax.axis_index("s")
        wid = sid * 2 + cid
        lanes = lax.iota(jnp.int32, 16)

        pltpu.sync_copy(src_hbm.at[pl.ds(wid * EC, EC)], src_v)
        pltpu.sync_copy(dst_hbm.at[pl.ds(wid * EC, EC)], dst_v)

        def fz(i, _):
            zbuf[pl.ds(i * 16, 16)] = jnp.zeros((16,), jnp.float32)
            return 0
        lax.fori_loop(0, 40, fz, 0)

        def fo(i, _):
            ones[pl.ds(i * 16, 16)] = jnp.ones((16,), jnp.float32)
            return 0
        lax.fori_loop(0, EC // 16, fo, 0)

        def fc(i, _):
            cnt[pl.ds(i * 16, 16)] = jnp.zeros((16,), jnp.int32)
            return 0
        lax.fori_loop(0, 32, fc, 0)

        pltpu.sync_copy(zbuf, degin_sh.at[pl.ds(sid * 640, 640)])
        pltpu.sync_copy(zbuf, degout_sh.at[pl.ds(sid * 640, 640)])
        plsc.subcore_barrier()

        cp_in = pltpu.async_copy(ones, degin_sh.at[dst_v], sem1, add=True)
        cp_out = pltpu.async_copy(ones, degout_sh.at[src_v], sem2, add=True)

        def hist(gi, _):
            dvec = dst_v[pl.ds(gi * 16, 16)]
            idx = (dvec // BKT) * 16 + lanes
            c = plsc.load_gather(cnt, [idx])
            plsc.store_scatter(cnt, [idx], c + 1)
            return 0
        lax.fori_loop(0, EC // 16, hist, 0)

        def scan(b2, total):
            row = cnt[pl.ds(b2 * 16, 16)]
            exc = plsc.cumsum(row) - row
            cur[pl.ds(b2 * 16, 16)] = exc + total
            return total + jnp.sum(row)
        lax.fori_loop(0, 32, scan, jnp.int32(0))

        offs_v[pl.ds(0, 16)] = plsc.load_gather(cur, [lanes * 16])
        offs_v[pl.ds(16, 16)] = plsc.load_gather(cur, [(lanes + 16) * 16])
        offs_v[pl.ds(32, 16)] = jnp.where(lanes == 0, EC, 0).astype(jnp.int32)

        def scat(gi, _):
            dvec = dst_v[pl.ds(gi * 16, 16)]
            svec = src_v[pl.ds(gi * 16, 16)]
            idx = (dvec // BKT) * 16 + lanes
            pos = plsc.load_gather(cur, [idx])
            plsc.store_scatter(cur, [idx], pos + 1)
            plsc.store_scatter(ssrc, [pos], svec)
            plsc.store_scatter(sdst, [pos], dvec)
            return 0
        lax.fori_loop(0, EC // 16, scat, 0)

        pltpu.sync_copy(ssrc, psrc_hbm.at[pl.ds(wid * RS, NPAD)])
        pltpu.sync_copy(sdst, pdst_hbm.at[pl.ds(wid * RS, NPAD)])
        pltpu.sync_copy(offs_v, offs_hbm.at[pl.ds(wid * 48, 48)])

        cp_in.wait()
        cp_out.wait()
        plsc.subcore_barrier()

        @pl.when(sid == 0)
        def _():
            pltpu.sync_copy(degin_sh, degin_hbm.at[pl.ds(cid * DEGP, DEGP)])
            pltpu.sync_copy(degout_sh, degout_hbm.at[pl.ds(cid * DEGP, DEGP)])

    return k(src, dst)


def _k2_segpass(psrc, pdst, offs, T):
    CH = 512
    mesh = plsc.VectorSubcoreMesh(core_axis_name="c", subcore_axis_name="s")

    @functools.partial(
        pl.kernel,
        out_type=jax.ShapeDtypeStruct((NPAD * 128,), jnp.float32),
        mesh=mesh,
        scratch_types=[
            pltpu.VMEM((32 * 48,), jnp.int32),
            pltpu.VMEM((330 * 128,), jnp.float32),
            pltpu.VMEM((CH,), jnp.int32),
            pltpu.VMEM((CH,), jnp.int32),
            pltpu.VMEM((CH + 16,), jnp.int32),
            pltpu.VMEM((CH, 128), jnp.float32),
            pltpu.VMEM((336,), jnp.int32),
            pltpu.SemaphoreType.DMA,
        ],
        compiler_params=pltpu.CompilerParams(needs_layout_passes=False),
    )
    def k(psrc_hbm, pdst_hbm, offs_hbm, T_hbm, SM_hbm,
          offs_v, acc, csrc, cdst, crow, rows, tag, sem):
        cid = lax.axis_index("c")
        sid = lax.axis_index("s")
        wid = sid * 2 + cid
        b = wid
        lanes = lax.iota(jnp.int32, 16)
        neg = jnp.full((16,), -jnp.inf, jnp.float32)
        zero = jnp.zeros((16,), jnp.float32)
        cvecs = [c * 16 + lanes for c in range(8)]

        pltpu.sync_copy(offs_hbm, offs_v)

        def fa(i, _):
            v = jnp.where((i % 8) < 4, zero, neg)
            acc[pl.ds(i * 16, 16)] = v
            return 0
        lax.fori_loop(0, 330 * 8, fa, 0)

        def seg(t2, _):
            v = offs_v[pl.ds(t2 * 48 + b, 16)]
            start = v[0]
            end = v[1]
            astart = (start // 8) * 8
            s0 = start - astart
            total = s0 + (end - start)
            nch = (total + CH - 1) // CH

            def chunk(j, _):
                off = t2 * RS + astart + j * CH
                pltpu.sync_copy(psrc_hbm.at[pl.ds(off, CH)], csrc)
                pltpu.sync_copy(pdst_hbm.at[pl.ds(off, CH)], cdst)

                def san(gi, _):
                    p = j * CH + gi * 16 + lanes
                    valid = (p >= s0) & (p < total)
                    sv = csrc[pl.ds(gi * 16, 16)]
                    dv = cdst[pl.ds(gi * 16, 16)]
                    csrc[pl.ds(gi * 16, 16)] = jnp.where(valid, sv, p & 8191)
                    crow[pl.ds(gi * 16, 16)] = jnp.where(
                        valid, dv - b * BKT, 313 + (p & 15))
                    return 0
                lax.fori_loop(0, CH // 16, san, 0)

                pltpu.async_copy(T_hbm.at[csrc], rows, sem).wait()

                def ap(gi, _):
                    gvec = crow[pl.ds(gi * 16, 16)]
                    evec = gi * 16 + lanes
                    ra = gvec * 128
                    vals = [plsc.load_gather(rows, [evec, cvecs[c]])
                            for c in range(8)]

                    def cond(committed):
                        return jnp.sum(jnp.where(committed, 0, 1)) > 0

                    def body(committed):
                        todo = jnp.logical_not(committed)
                        plsc.store_scatter(tag, [gvec], lanes, mask=todo)
                        w = plsc.load_gather(tag, [gvec])
                        winner = (w == lanes) & todo
                        for c in range(8):
                            idx = ra + cvecs[c]
                            cur = plsc.load_gather(acc, [idx])
                            new = (cur + vals[c]) if c < 4 \
                                else jnp.maximum(cur, vals[c])
                            plsc.store_scatter(acc, [idx], new, mask=winner)
                        return committed | winner

                    lax.while_loop(cond, body, jnp.zeros((16,), jnp.bool_))
                    return 0
                lax.fori_loop(0, CH // 16, ap, 0)
                return 0

            lax.fori_loop(0, nch, chunk, 0)
            return 0

        lax.fori_loop(0, 32, seg, 0)

        pltpu.sync_copy(acc.at[pl.ds(0, BKT * 128)],
                        SM_hbm.at[pl.ds(b * BKT * 128, BKT * 128)])

    return k(psrc, pdst, offs, T)


def _k3_dot(src, dst, h):
    CH = 80
    NCH = EC // CH
    mesh = plsc.VectorSubcoreMesh(core_axis_name="c", subcore_axis_name="s")

    @functools.partial(
        pl.kernel,
        out_type=jax.ShapeDtypeStruct((E,), jnp.float32),
        mesh=mesh,
        scratch_types=[
            pltpu.VMEM((CH,), jnp.int32),
            pltpu.VMEM((CH,), jnp.int32),
            pltpu.VMEM((CH,), jnp.int32),
            pltpu.VMEM((CH,), jnp.int32),
            pltpu.VMEM((CH, 128), jnp.float32),
            pltpu.VMEM((CH, 128), jnp.float32),
            pltpu.VMEM((CH, 128), jnp.float32),
            pltpu.VMEM((CH, 128), jnp.float32),
            pltpu.VMEM((CH,), jnp.float32),
            pltpu.SemaphoreType.DMA,
            pltpu.SemaphoreType.DMA,
            pltpu.SemaphoreType.DMA,
            pltpu.SemaphoreType.DMA,
        ],
        compiler_params=pltpu.CompilerParams(needs_layout_passes=False),
    )
    def k(src_hbm, dst_hbm, h_hbm, out_hbm, csrc0, csrc1, cdst0, cdst1,
          U0, U1, V0, V1, outv, sU0, sU1, sV0, sV1):
        cid = lax.axis_index("c")
        sid = lax.axis_index("s")
        wid = sid * 2 + cid
        base = wid * EC
        lanes = lax.iota(jnp.int32, 16)
        CS = (csrc0, csrc1)
        CD = (cdst0, cdst1)
        UU = (U0, U1)
        VV = (V0, V1)
        semU = (sU0, sU1)
        semV = (sV0, sV1)

        def issue(j, slot):
            pltpu.sync_copy(src_hbm.at[pl.ds(base + j * CH, CH)], CS[slot])
            pltpu.sync_copy(dst_hbm.at[pl.ds(base + j * CH, CH)], CD[slot])
            pltpu.async_copy(h_hbm.at[CS[slot]], UU[slot], semU[slot])
            pltpu.async_copy(h_hbm.at[CD[slot]], VV[slot], semV[slot])

        def wait(slot):
            pltpu.make_async_copy(h_hbm.at[CS[slot]], UU[slot], semU[slot]).wait()
            pltpu.make_async_copy(h_hbm.at[CD[slot]], VV[slot], semV[slot]).wait()

        def compute(j, slot):
            Ur, Vr = UU[slot], VV[slot]

            def group(gi, _):
                res = jnp.zeros((16,), jnp.float32)
                for i in range(16):
                    e = gi * 16 + i
                    d = Ur[e, pl.ds(0, 16)] * Vr[e, pl.ds(0, 16)]
                    for c in range(1, 8):
                        d = d + Ur[e, pl.ds(c * 16, 16)] * Vr[e, pl.ds(c * 16, 16)]
                    res = jnp.where(lanes == i, jnp.sum(d), res)
                outv[pl.ds(gi * 16, 16)] = res
                return 0

            lax.fori_loop(0, CH // 16, group, 0)
            pltpu.sync_copy(outv, out_hbm.at[pl.ds(base + j * CH, CH)])

        issue(0, 0)

        def pair(i, _):
            j0 = 2 * i
            wait(0)
            issue(j0 + 1, 1)
            compute(j0, 0)
            wait(1)

            @pl.when(j0 + 2 < NCH)
            def _():
                issue(j0 + 2, 0)

            compute(j0 + 1, 1)
            return 0

        lax.fori_loop(0, (NCH - 1) // 2, pair, 0)
        wait(0)
        compute(NCH - 1, 0)

    return k(src, dst, h)


def kernel(g, x_n, x_e, Wn1, bn1, Wn2, bn2, Th1, Tb1, Ph1, Pb1, Th2, Tb2, Ph2, Pb2):
    src = g[0].reshape(E)
    dst = g[1].reshape(E)

    psrc, pdst, offs, degin_p, degout_p = _k1_partition(src, dst)

    T1, C1, norm_s, norm_d, deg_in = _tca(
        degout_p.reshape(2, DEGP), degin_p.reshape(2, DEGP), x_n, x_e, Wn1,
        bn1.reshape(1, 64), Th1, Tb1.reshape(1, 64), Ph1, Pb1.reshape(1, 64))

    SM1 = _k2_segpass(psrc, pdst, offs, T1).reshape(NPAD, 128)

    T2, C2 = _tcb(SM1, C1, deg_in, norm_s, norm_d, bn1.reshape(1, 64), Wn2,
                  Th2, Tb2.reshape(1, 64), Ph2, Pb2.reshape(1, 64))

    SM2 = _k2_segpass(psrc, pdst, offs, T2).reshape(NPAD, 128)

    h = _tcc(SM2, C2, deg_in, norm_d, bn2.reshape(1, 64))

    score = _k3_dot(src, dst, h)
    return score.reshape(E, 1)
